# baseline (device time: 30905 ns/iter reference)
import jax
import jax.numpy as jnp
from jax import lax
from jax.experimental import pallas as pl
from jax.experimental.pallas import tpu as pltpu

N_DEV = 4


def kernel(Q, K, V):
    b, s, h, d = Q.shape
    bh = b * h
    scale = d ** -0.5

    q3 = (
        (jnp.transpose(Q, (0, 2, 1, 3)) * scale)
        .reshape(bh, s, d)
        .astype(jnp.bfloat16)
    )

    kt = jnp.transpose(K, (0, 2, 3, 1)).reshape(bh, d, s)
    vn = jnp.transpose(V, (0, 2, 1, 3)).reshape(bh, s, d)

    def enc_scale(sc):
        q14 = jnp.round(sc * 65536.0).astype(jnp.int32)
        return (q14 // 127).astype(jnp.int8), (q14 % 127).astype(jnp.int8)

    ks = jnp.max(jnp.abs(kt), axis=1, keepdims=True) / 127.0
    k8 = jnp.clip(jnp.round(kt / ks), -127, 127).astype(jnp.int8)
    ka, kb2 = enc_scale(ks)

    vs = jnp.max(jnp.abs(vn), axis=2, keepdims=True) / 127.0
    v8 = jnp.clip(jnp.round(vn / vs), -127, 127).astype(jnp.int8)
    va, vb2 = enc_scale(jnp.transpose(vs, (0, 2, 1)))

    k3 = jnp.concatenate([k8, ka, kb2, va, vb2], axis=1)
    dk = d + 4

    def body(q_ref, k_ref, v_ref, out_ref, kbuf, vbuf, ss, rs):
        my = lax.axis_index("i")
        left = (my - 1) % N_DEV
        right = (my + 1) % N_DEV

        barrier = pltpu.get_barrier_semaphore()
        for nbr in (left, right):
            pl.semaphore_signal(
                barrier, inc=1, device_id=(nbr,),
                device_id_type=pl.DeviceIdType.MESH,
            )
        pl.semaphore_wait(barrier, 2)

        kbuf[0] = k_ref[...]
        vbuf[0] = v_ref[...]

        def rdma(src, dst, i, dev):
            return pltpu.make_async_remote_copy(
                src_ref=src, dst_ref=dst,
                send_sem=ss.at[i], recv_sem=rs.at[i],
                device_id=(dev,), device_id_type=pl.DeviceIdType.MESH,
            )

        k_r = rdma(kbuf.at[0], kbuf.at[1], 0, right)
        v_r = rdma(vbuf.at[0], vbuf.at[1], 1, right)
        k_l = rdma(kbuf.at[0], kbuf.at[2], 2, left)
        v_l = rdma(vbuf.at[0], vbuf.at[2], 3, left)
        for r in (k_r, v_r, k_l, v_l):
            r.start()

        def update(slot, accs, ls):
            new_accs, new_ls = [], []
            for i in range(bh):
                ka_ = kbuf[slot, i, d:d + 1, :].astype(jnp.float32)
                kb_ = kbuf[slot, i, d + 1:d + 2, :].astype(jnp.float32)
                va_ = kbuf[slot, i, d + 2:d + 3, :].astype(jnp.float32)
                vb_ = kbuf[slot, i, d + 3:d + 4, :].astype(jnp.float32)
                ksr = (ka_ * 127.0 + kb_) * (1.0 / 65536.0)
                vsr = (va_ * 127.0 + vb_) * (1.0 / 65536.0)
                s_int = lax.dot_general(
                    q_ref[i], kbuf[slot, i, 0:d, :].astype(jnp.bfloat16),
                    (((1,), (0,)), ((), ())),
                    preferred_element_type=jnp.float32,
                )
                p = jnp.exp((s_int * ksr).astype(jnp.bfloat16))
                lsum = jnp.sum(p, axis=1, dtype=jnp.float32, keepdims=True)
                vsc = (
                    jnp.transpose(vsr, (1, 0))
                    * vbuf[slot, i].astype(jnp.float32)
                ).astype(jnp.bfloat16)
                pv = lax.dot_general(
                    p, vsc,
                    (((1,), (0,)), ((), ())),
                    preferred_element_type=jnp.float32,
                )
                if accs is None:
                    new_accs.append(pv)
                    new_ls.append(lsum)
                else:
                    new_accs.append(accs[i] + pv)
                    new_ls.append(ls[i] + lsum)
            return new_accs, new_ls

        accs, ls = update(0, None, None)

        k_r.wait_recv()
        k_f = rdma(kbuf.at[1], kbuf.at[3], 4, right)
        k_f.start()
        v_l.wait_recv()
        v_f = rdma(vbuf.at[2], vbuf.at[3], 5, left)
        v_f.start()

        v_r.wait_recv()
        accs, ls = update(1, accs, ls)
        k_l.wait_recv()
        accs, ls = update(2, accs, ls)

        k_f.wait_recv()
        v_f.wait_recv()
        accs, ls = update(3, accs, ls)

        for i in range(bh):
            out_ref[i] = accs[i] / ls[i]

        for r in (k_r, v_r, k_l, v_l, k_f, v_f):
            r.wait_send()

    params_cls = getattr(pltpu, "CompilerParams", None) or pltpu.TPUCompilerParams
    out = pl.pallas_call(
        body,
        out_shape=jax.ShapeDtypeStruct((bh, s, d), jnp.float32),
        in_specs=[pl.BlockSpec(memory_space=pltpu.VMEM)] * 3,
        out_specs=pl.BlockSpec(memory_space=pltpu.VMEM),
        scratch_shapes=[
            pltpu.VMEM((N_DEV, bh, dk, s), jnp.int8),
            pltpu.VMEM((N_DEV, bh, s, d), jnp.int8),
            pltpu.SemaphoreType.DMA((6,)),
            pltpu.SemaphoreType.DMA((6,)),
        ],
        compiler_params=params_cls(collective_id=0),
    )(q3, k3, v8)

    return out.reshape(b, h, s, d).transpose(0, 2, 1, 3)


# device time: 29986 ns/iter; 1.0306x vs baseline; 1.0306x over previous
import jax
import jax.numpy as jnp
from jax import lax
from jax.experimental import pallas as pl
from jax.experimental.pallas import tpu as pltpu

N_DEV = 4


def kernel(Q, K, V):
    b, s, h, d = Q.shape
    bh = b * h
    scale = d ** -0.5

    q3 = (
        (jnp.transpose(Q, (0, 2, 1, 3)) * scale)
        .reshape(bh, s, d)
        .astype(jnp.bfloat16)
    )

    kt = jnp.transpose(K, (0, 2, 3, 1)).reshape(bh, d, s)
    vn = jnp.transpose(V, (0, 2, 1, 3)).reshape(bh, s, d)

    def enc_scale(sc):
        q14 = jnp.round(sc * 65536.0).astype(jnp.int32)
        return (q14 // 127).astype(jnp.int8), (q14 % 127).astype(jnp.int8)

    ks = jnp.max(jnp.abs(kt), axis=1, keepdims=True) / 127.0
    k8 = jnp.clip(jnp.round(kt / ks), -127, 127).astype(jnp.int8)
    ka, kb2 = enc_scale(ks)

    vs = jnp.max(jnp.abs(vn), axis=2, keepdims=True) / 127.0
    v8 = jnp.clip(jnp.round(vn / vs), -127, 127).astype(jnp.int8)
    va, vb2 = enc_scale(jnp.transpose(vs, (0, 2, 1)))

    k3 = jnp.concatenate([k8, ka, kb2, va, vb2], axis=1)
    dk = d + 4

    def body(q_ref, k_ref, v_ref, out_ref, kbuf, vbuf, ss, rs):
        my = lax.axis_index("i")
        left = (my - 1) % N_DEV
        right = (my + 1) % N_DEV

        barrier = pltpu.get_barrier_semaphore()
        for nbr in (left, right):
            pl.semaphore_signal(
                barrier, inc=1, device_id=(nbr,),
                device_id_type=pl.DeviceIdType.MESH,
            )
        pl.semaphore_wait(barrier, 2)

        kbuf[0] = k_ref[...]
        vbuf[0] = v_ref[...]

        def rdma(src, dst, i, dev):
            return pltpu.make_async_remote_copy(
                src_ref=src, dst_ref=dst,
                send_sem=ss.at[i], recv_sem=rs.at[i],
                device_id=(dev,), device_id_type=pl.DeviceIdType.MESH,
            )

        k_r = rdma(kbuf.at[0], kbuf.at[1], 0, right)
        v_r = rdma(vbuf.at[0], vbuf.at[1], 1, right)
        k_l = rdma(kbuf.at[0], kbuf.at[2], 2, left)
        v_l = rdma(vbuf.at[0], vbuf.at[2], 3, left)
        for r in (k_r, v_r, k_l, v_l):
            r.start()

        def update(slot, accs, ls):
            new_accs, new_ls = [], []
            for i in range(bh):
                ka_ = kbuf[slot, i, d:d + 1, :].astype(jnp.float32)
                kb_ = kbuf[slot, i, d + 1:d + 2, :].astype(jnp.float32)
                va_ = kbuf[slot, i, d + 2:d + 3, :].astype(jnp.float32)
                vb_ = kbuf[slot, i, d + 3:d + 4, :].astype(jnp.float32)
                ksr = (ka_ * 127.0 + kb_) * (1.0 / 65536.0)
                vsr = (va_ * 127.0 + vb_) * (1.0 / 65536.0)
                s_int = lax.dot_general(
                    q_ref[i], kbuf[slot, i, 0:d, :].astype(jnp.bfloat16),
                    (((1,), (0,)), ((), ())),
                    preferred_element_type=jnp.float32,
                )
                p = jnp.exp(s_int * ksr)
                lsum = jnp.sum(p, axis=1, keepdims=True)
                pv = lax.dot_general(
                    (p * vsr).astype(jnp.bfloat16),
                    vbuf[slot, i].astype(jnp.bfloat16),
                    (((1,), (0,)), ((), ())),
                    preferred_element_type=jnp.float32,
                )
                if accs is None:
                    new_accs.append(pv)
                    new_ls.append(lsum)
                else:
                    new_accs.append(accs[i] + pv)
                    new_ls.append(ls[i] + lsum)
            return new_accs, new_ls

        accs, ls = update(0, None, None)

        k_r.wait_recv()
        k_f = rdma(kbuf.at[1], kbuf.at[3], 4, right)
        k_f.start()
        v_l.wait_recv()
        v_f = rdma(vbuf.at[2], vbuf.at[3], 5, left)
        v_f.start()

        v_r.wait_recv()
        accs, ls = update(1, accs, ls)
        k_l.wait_recv()
        accs, ls = update(2, accs, ls)

        k_f.wait_recv()
        v_f.wait_recv()
        accs, ls = update(3, accs, ls)

        for i in range(bh):
            out_ref[i] = accs[i] / ls[i]

        for r in (k_r, v_r, k_l, v_l, k_f, v_f):
            r.wait_send()

    params_cls = getattr(pltpu, "CompilerParams", None) or pltpu.TPUCompilerParams
    out = pl.pallas_call(
        body,
        out_shape=jax.ShapeDtypeStruct((bh, s, d), jnp.float32),
        in_specs=[pl.BlockSpec(memory_space=pltpu.VMEM)] * 3,
        out_specs=pl.BlockSpec(memory_space=pltpu.VMEM),
        scratch_shapes=[
            pltpu.VMEM((N_DEV, bh, dk, s), jnp.int8),
            pltpu.VMEM((N_DEV, bh, s, d), jnp.int8),
            pltpu.SemaphoreType.DMA((6,)),
            pltpu.SemaphoreType.DMA((6,)),
        ],
        compiler_params=params_cls(collective_id=0),
    )(q3, k3, v8)

    return out.reshape(b, h, s, d).transpose(0, 2, 1, 3)
